# baseline (speedup 1.0000x reference)
import functools

import jax
import jax.numpy as jnp
from jax.experimental import pallas as pl
from jax.experimental.pallas import tpu as pltpu

_BN_EPS = 1e-5
_T0 = (((0,), (0,)), ((), ()))
_T1 = (((1,), (1,)), ((), ()))


def _encoder_body(sxt_ref, at_ref, xt_ref, we_ref, ep_ref, zt_ref, st_ref):
    k, fs, w = sxt_ref.shape
    lat = we_ref.shape[1]
    fx = xt_ref.shape[0]
    b_col = ep_ref[:, 0:1]
    zacc = None
    for i in range(k):
        agg = None
        for j in range(k):
            term = at_ref[i, j:j + 1, :] * sxt_ref[j]
            agg = term if agg is None else agg + term
        h = jax.lax.dot_general(we_ref[...], agg, _T0,
                                preferred_element_type=jnp.float32)
        h = jnp.maximum(h + b_col, 0.0)
        zacc = h if zacc is None else zacc + h
    zt = zacc * (1.0 / k)
    zt_ref[...] = zt
    xg = xt_ref[...]
    pad = jnp.zeros((lat - fx, 1), jnp.float32)
    st = jnp.concatenate(
        [
            jnp.sum(zt, axis=1, keepdims=True),
            jnp.sum(zt * zt, axis=1, keepdims=True),
            jnp.concatenate([jnp.sum(xg, axis=1, keepdims=True), pad], axis=0),
            jnp.concatenate([jnp.sum(xg * xg, axis=1, keepdims=True), pad], axis=0),
            jnp.zeros((lat, 4), jnp.float32),
        ],
        axis=1,
    )
    st_ref[...] = st.reshape(1, lat, 8)


def _gnn_body(xt_ref, zt_ref, at_ref, st_ref, gb_ref, w1x_ref, w1z_ref,
              b1_ref, w2t_ref, outt_ref, *, total_n):
    fx, m = xt_ref.shape
    lat = zt_ref.shape[0]
    ng = at_ref.shape[0]
    g = at_ref.shape[2]
    hid = w1x_ref.shape[1]
    d_out = w2t_ref.shape[0]
    tot = jnp.sum(st_ref[...], axis=0)
    inv_n = 1.0 / total_n
    mu_z = tot[:, 0:1] * inv_n
    var_z = tot[:, 1:2] * inv_n - mu_z * mu_z
    mu_x = tot[:fx, 2:3] * inv_n
    var_x = tot[:fx, 3:4] * inv_n - mu_x * mu_x
    sc_x = gb_ref[:fx, 0:1] * jax.lax.rsqrt(var_x + _BN_EPS)
    sc_z = gb_ref[:, 1:2] * jax.lax.rsqrt(var_z + _BN_EPS)
    sh_x = gb_ref[:fx, 2:3] - mu_x * sc_x
    sh_z = gb_ref[:, 3:4] - mu_z * sc_z
    xn = xt_ref[...] * sc_x + sh_x
    zn = zt_ref[...] * sc_z + sh_z
    pre = (
        jax.lax.dot_general(xn, w1x_ref[...], _T0,
                            preferred_element_type=jnp.float32)
        + jax.lax.dot_general(zn, w1z_ref[...], _T0,
                              preferred_element_type=jnp.float32)
    ).reshape(g, ng, hid)
    adj = jnp.transpose(at_ref[...], (2, 0, 1))
    h = jnp.einsum('gij,gjf->gif', adj, pre,
                   preferred_element_type=jnp.float32)
    h = jnp.maximum(h + b1_ref[...].reshape(1, 1, hid), 0.0)
    pooled = jnp.mean(h, axis=1)
    outt_ref[...] = (
        jax.lax.dot_general(w2t_ref[...], pooled, _T1,
                            preferred_element_type=jnp.float32)
        + gb_ref[:d_out, 4:5]
    )


def _pick_block(total, preferred):
    for cand in (preferred, preferred // 2, preferred // 4, 128, 64, 32, 16, 8):
        if cand and total % cand == 0:
            return cand
    return total


def _pad_rows(v, rows):
    return jnp.pad(v, (0, rows - v.shape[0]))


@functools.partial(jax.jit, static_argnames=())
def kernel(sub_x, a_sub_blocks, x, a_blocks, w_enc, b_enc, gamma, beta,
           w1x, w1z, b1, w2, b2):
    n, k, fs = sub_x.shape
    b, ng, _ = a_blocks.shape
    fx = x.shape[1]
    lat = w_enc.shape[1]
    hid = w1x.shape[1]
    d_out = w2.shape[1]

    sxt = jnp.transpose(sub_x, (1, 2, 0))
    at = jnp.transpose(a_sub_blocks, (1, 2, 0))
    xt = x.T
    w2t = w2.T

    ep = jnp.concatenate(
        [b_enc.T, jnp.zeros((lat, 7), jnp.float32)], axis=1)

    wb = _pick_block(n, 2048)
    nblk = n // wb
    zt, stats = pl.pallas_call(
        _encoder_body,
        out_shape=[
            jax.ShapeDtypeStruct((lat, n), jnp.float32),
            jax.ShapeDtypeStruct((nblk, lat, 8), jnp.float32),
        ],
        grid=(nblk,),
        in_specs=[
            pl.BlockSpec((k, fs, wb), lambda i: (0, 0, i)),
            pl.BlockSpec((k, k, wb), lambda i: (0, 0, i)),
            pl.BlockSpec((fx, wb), lambda i: (0, i)),
            pl.BlockSpec((fs, lat), lambda i: (0, 0)),
            pl.BlockSpec((lat, 8), lambda i: (0, 0)),
        ],
        out_specs=[
            pl.BlockSpec((lat, wb), lambda i: (0, i)),
            pl.BlockSpec((1, lat, 8), lambda i: (i, 0, 0)),
        ],
        compiler_params=pltpu.CompilerParams(
            dimension_semantics=("parallel",),
            vmem_limit_bytes=100 * 1024 * 1024,
        ),
        cost_estimate=pl.CostEstimate(
            flops=int(2 * n * k * lat * (fs + k)),
            transcendentals=0,
            bytes_accessed=int(
                (sub_x.size + a_sub_blocks.size + x.size + n * lat) * 4),
        ),
    )(sxt, at, xt, w_enc, ep)

    gb2 = jnp.stack(
        [
            _pad_rows(gamma[0, :fx], lat), gamma[0, fx:],
            _pad_rows(beta[0, :fx], lat), beta[0, fx:],
            _pad_rows(b2[0], lat), jnp.zeros((lat,), jnp.float32),
            jnp.zeros((lat,), jnp.float32), jnp.zeros((lat,), jnp.float32),
        ],
        axis=1,
    )

    gb = _pick_block(b, 128)
    nblk_s = stats.shape[0]
    outt = pl.pallas_call(
        functools.partial(_gnn_body, total_n=n),
        out_shape=jax.ShapeDtypeStruct((d_out, b), jnp.float32),
        grid=(b // gb,),
        in_specs=[
            pl.BlockSpec((fx, gb * ng), lambda i: (0, i)),
            pl.BlockSpec((lat, gb * ng), lambda i: (0, i)),
            pl.BlockSpec((ng, ng, gb), lambda i: (0, 0, i)),
            pl.BlockSpec((nblk_s, lat, 8), lambda i: (0, 0, 0)),
            pl.BlockSpec((lat, 8), lambda i: (0, 0)),
            pl.BlockSpec((fx, hid), lambda i: (0, 0)),
            pl.BlockSpec((lat, hid), lambda i: (0, 0)),
            pl.BlockSpec((1, hid), lambda i: (0, 0)),
            pl.BlockSpec((d_out, hid), lambda i: (0, 0)),
        ],
        out_specs=pl.BlockSpec((d_out, gb), lambda i: (0, i)),
        compiler_params=pltpu.CompilerParams(
            dimension_semantics=("parallel",),
            vmem_limit_bytes=100 * 1024 * 1024,
        ),
        cost_estimate=pl.CostEstimate(
            flops=int(2 * b * ng * ((fx + lat) * hid + ng * hid)
                      + 2 * b * hid * d_out),
            transcendentals=0,
            bytes_accessed=int(
                (b * ng * (fx + lat + ng) + b * d_out) * 4),
        ),
    )(xt, zt, jnp.transpose(a_blocks, (1, 2, 0)), stats, gb2,
      w1x, w1z, b1, w2t)

    return outt.T


# speedup vs baseline: 1.0527x; 1.0527x over previous
import functools

import jax
import jax.numpy as jnp
from jax.experimental import pallas as pl
from jax.experimental.pallas import tpu as pltpu

_BN_EPS = 1e-5
_T0 = (((0,), (0,)), ((), ()))
_T1 = (((1,), (1,)), ((), ()))


def _encoder_body(sxt_ref, at_ref, xt_ref, we_ref, ep_ref, zt_ref, st_ref):
    k, fs, w = sxt_ref.shape
    lat = we_ref.shape[1]
    fx = xt_ref.shape[0]
    b_col = ep_ref[:, 0:1]
    zacc = None
    for i in range(k):
        agg = None
        for j in range(k):
            term = at_ref[i, j:j + 1, :] * sxt_ref[j]
            agg = term if agg is None else agg + term
        h = jax.lax.dot_general(we_ref[...], agg, _T0,
                                preferred_element_type=jnp.float32)
        h = jnp.maximum(h + b_col, 0.0)
        zacc = h if zacc is None else zacc + h
    zt = zacc * (1.0 / k)
    zt_ref[...] = zt
    xg = xt_ref[...]
    pad = jnp.zeros((lat - fx, 1), jnp.float32)
    st = jnp.concatenate(
        [
            jnp.sum(zt, axis=1, keepdims=True),
            jnp.sum(zt * zt, axis=1, keepdims=True),
            jnp.concatenate([jnp.sum(xg, axis=1, keepdims=True), pad], axis=0),
            jnp.concatenate([jnp.sum(xg * xg, axis=1, keepdims=True), pad], axis=0),
            jnp.zeros((lat, 4), jnp.float32),
        ],
        axis=1,
    )
    st_ref[...] = st.reshape(1, lat, 8)


def _gnn_body(xt_ref, zt_ref, at_ref, st_ref, gb_ref, w1x_ref, w1z_ref,
              b1_ref, w2t_ref, outt_ref, *, total_n):
    fx, m = xt_ref.shape
    lat = zt_ref.shape[0]
    ng = at_ref.shape[0]
    g = at_ref.shape[2]
    hid = w1x_ref.shape[1]
    d_out = w2t_ref.shape[0]
    tot = jnp.sum(st_ref[...], axis=0)
    inv_n = 1.0 / total_n
    mu_z = tot[:, 0:1] * inv_n
    var_z = tot[:, 1:2] * inv_n - mu_z * mu_z
    mu_x = tot[:fx, 2:3] * inv_n
    var_x = tot[:fx, 3:4] * inv_n - mu_x * mu_x
    sc_x = gb_ref[:fx, 0:1] * jax.lax.rsqrt(var_x + _BN_EPS)
    sc_z = gb_ref[:, 1:2] * jax.lax.rsqrt(var_z + _BN_EPS)
    sh_x = gb_ref[:fx, 2:3] - mu_x * sc_x
    sh_z = gb_ref[:, 3:4] - mu_z * sc_z
    xn = xt_ref[...] * sc_x + sh_x
    zn = zt_ref[...] * sc_z + sh_z
    pre = (
        jax.lax.dot_general(xn, w1x_ref[...], _T0,
                            preferred_element_type=jnp.float32)
        + jax.lax.dot_general(zn, w1z_ref[...], _T0,
                              preferred_element_type=jnp.float32)
    ).reshape(g, ng, hid)
    adj = jnp.transpose(at_ref[...], (2, 0, 1))
    h = jnp.einsum('gij,gjf->gif', adj, pre,
                   preferred_element_type=jnp.float32)
    h = jnp.maximum(h + b1_ref[...].reshape(1, 1, hid), 0.0)
    pooled = jnp.mean(h, axis=1)
    outt_ref[...] = (
        jax.lax.dot_general(w2t_ref[...], pooled, _T1,
                            preferred_element_type=jnp.float32)
        + gb_ref[:d_out, 4:5]
    )


def _pick_block(total, preferred):
    for cand in (preferred, preferred // 2, preferred // 4, 128, 64, 32, 16, 8):
        if cand and total % cand == 0:
            return cand
    return total


def _pad_rows(v, rows):
    return jnp.pad(v, (0, rows - v.shape[0]))


@functools.partial(jax.jit, static_argnames=())
def kernel(sub_x, a_sub_blocks, x, a_blocks, w_enc, b_enc, gamma, beta,
           w1x, w1z, b1, w2, b2):
    n, k, fs = sub_x.shape
    b, ng, _ = a_blocks.shape
    fx = x.shape[1]
    lat = w_enc.shape[1]
    hid = w1x.shape[1]
    d_out = w2.shape[1]

    sxt = jnp.transpose(sub_x, (1, 2, 0))
    at = jnp.transpose(a_sub_blocks, (1, 2, 0))
    xt = x.T
    w2t = w2.T

    ep = jnp.concatenate(
        [b_enc.T, jnp.zeros((lat, 7), jnp.float32)], axis=1)

    wb = _pick_block(n, 2048)
    nblk = n // wb
    zt, stats = pl.pallas_call(
        _encoder_body,
        out_shape=[
            jax.ShapeDtypeStruct((lat, n), jnp.float32),
            jax.ShapeDtypeStruct((nblk, lat, 8), jnp.float32),
        ],
        grid=(nblk,),
        in_specs=[
            pl.BlockSpec((k, fs, wb), lambda i: (0, 0, i)),
            pl.BlockSpec((k, k, wb), lambda i: (0, 0, i)),
            pl.BlockSpec((fx, wb), lambda i: (0, i)),
            pl.BlockSpec((fs, lat), lambda i: (0, 0)),
            pl.BlockSpec((lat, 8), lambda i: (0, 0)),
        ],
        out_specs=[
            pl.BlockSpec((lat, wb), lambda i: (0, i)),
            pl.BlockSpec((1, lat, 8), lambda i: (i, 0, 0)),
        ],
        compiler_params=pltpu.CompilerParams(
            dimension_semantics=("parallel",),
            vmem_limit_bytes=100 * 1024 * 1024,
        ),
        cost_estimate=pl.CostEstimate(
            flops=int(2 * n * k * lat * (fs + k)),
            transcendentals=0,
            bytes_accessed=int(
                (sub_x.size + a_sub_blocks.size + x.size + n * lat) * 4),
        ),
    )(sxt, at, xt, w_enc, ep)

    gb2 = jnp.stack(
        [
            _pad_rows(gamma[0, :fx], lat), gamma[0, fx:],
            _pad_rows(beta[0, :fx], lat), beta[0, fx:],
            _pad_rows(b2[0], lat), jnp.zeros((lat,), jnp.float32),
            jnp.zeros((lat,), jnp.float32), jnp.zeros((lat,), jnp.float32),
        ],
        axis=1,
    )

    gb = _pick_block(b, 256)
    nblk_s = stats.shape[0]
    outt = pl.pallas_call(
        functools.partial(_gnn_body, total_n=n),
        out_shape=jax.ShapeDtypeStruct((d_out, b), jnp.float32),
        grid=(b // gb,),
        in_specs=[
            pl.BlockSpec((fx, gb * ng), lambda i: (0, i)),
            pl.BlockSpec((lat, gb * ng), lambda i: (0, i)),
            pl.BlockSpec((ng, ng, gb), lambda i: (0, 0, i)),
            pl.BlockSpec((nblk_s, lat, 8), lambda i: (0, 0, 0)),
            pl.BlockSpec((lat, 8), lambda i: (0, 0)),
            pl.BlockSpec((fx, hid), lambda i: (0, 0)),
            pl.BlockSpec((lat, hid), lambda i: (0, 0)),
            pl.BlockSpec((1, hid), lambda i: (0, 0)),
            pl.BlockSpec((d_out, hid), lambda i: (0, 0)),
        ],
        out_specs=pl.BlockSpec((d_out, gb), lambda i: (0, i)),
        compiler_params=pltpu.CompilerParams(
            dimension_semantics=("parallel",),
            vmem_limit_bytes=100 * 1024 * 1024,
        ),
        cost_estimate=pl.CostEstimate(
            flops=int(2 * b * ng * ((fx + lat) * hid + ng * hid)
                      + 2 * b * hid * d_out),
            transcendentals=0,
            bytes_accessed=int(
                (b * ng * (fx + lat + ng) + b * d_out) * 4),
        ),
    )(xt, zt, jnp.transpose(a_blocks, (1, 2, 0)), stats, gb2,
      w1x, w1z, b1, w2t)

    return outt.T
